# baseline (speedup 1.0000x reference)
import functools

import jax
import jax.numpy as jnp
from jax import lax
from jax.experimental import pallas as pl
from jax.experimental.pallas import tpu as pltpu
from jax.experimental.pallas import tpu_sc as plsc

_NC = 2
_NS = 16
_NW = _NC * _NS


@functools.partial(jax.jit, static_argnames=("nb", "s", "d"))
def _gather_rows(table, idx2d, nb, s, d):
    b_per_w = nb // _NW
    assert b_per_w % 2 == 0
    n_pairs = b_per_w // 2
    mesh = plsc.VectorSubcoreMesh(
        core_axis_name="c", subcore_axis_name="s",
        num_cores=_NC, num_subcores=_NS)

    @functools.partial(
        pl.kernel,
        out_type=jax.ShapeDtypeStruct((nb, s, d), jnp.float32),
        mesh=mesh,
        scratch_types=[
            pltpu.VMEM((b_per_w, s), jnp.int32),
            pltpu.VMEM((2, s, d), jnp.float32),
            pltpu.SemaphoreType.DMA((2,)),
            pltpu.SemaphoreType.DMA((2,)),
        ],
        compiler_params=pltpu.CompilerParams(use_tc_tiling_on_sc=False),
    )
    def run(table_hbm, idx_hbm, out_hbm, idx_v, rows_v, gsem, ssem):
        wid = lax.axis_index("s") * _NC + lax.axis_index("c")
        base = wid * b_per_w
        pltpu.sync_copy(idx_hbm.at[pl.ds(base, b_per_w)], idx_v)

        def gather(buf, k):
            return pltpu.make_async_copy(
                table_hbm.at[idx_v.at[k]], rows_v.at[buf], gsem.at[buf])

        def store(buf, k):
            return pltpu.make_async_copy(
                rows_v.at[buf], out_hbm.at[base + k], ssem.at[buf])

        gather(0, 0).start()
        gather(1, 1).start()

        @pl.loop(0, n_pairs)
        def _pair(g):
            k0 = 2 * g
            k1 = k0 + 1
            last = b_per_w - 1
            k2 = jnp.minimum(k0 + 2, last)
            k3 = jnp.minimum(k0 + 3, last)
            gather(0, k0).wait()
            store(0, k0).start()
            gather(1, k1).wait()
            store(1, k1).start()
            store(0, k0).wait()
            gather(0, k2).start()
            store(1, k1).wait()
            gather(1, k3).start()

        gather(0, b_per_w - 1).wait()
        gather(1, b_per_w - 1).wait()

    return run(table, idx2d)


def kernel(idx, logits_table):
    nb, s = idx.shape
    v, d = logits_table.shape
    del v
    return _gather_rows(logits_table, idx.astype(jnp.int32), nb, s, d)


# speedup vs baseline: 1.3799x; 1.3799x over previous
import functools

import jax
import jax.numpy as jnp
from jax import lax
from jax.experimental import pallas as pl
from jax.experimental.pallas import tpu as pltpu
from jax.experimental.pallas import tpu_sc as plsc

_NC = 2
_NS = 16
_NW = _NC * _NS
_CHUNK = 40
_DPAD = 1024
_IPAD = 128


@functools.partial(jax.jit, static_argnames=("n",))
def _gather_rows(table, idx2d, n):
    b_per_w = n // _NW
    n_chunks = b_per_w // _CHUNK
    assert n_chunks % 2 == 0 and b_per_w % _CHUNK == 0
    n_pairs = n_chunks // 2
    row_pad = idx2d.shape[1]
    mesh = plsc.VectorSubcoreMesh(
        core_axis_name="c", subcore_axis_name="s",
        num_cores=_NC, num_subcores=_NS)

    @functools.partial(
        pl.kernel,
        out_type=jax.ShapeDtypeStruct((n, _DPAD), jnp.float32),
        mesh=mesh,
        scratch_types=[
            pltpu.VMEM((row_pad,), jnp.int32),
            pltpu.VMEM((2, _CHUNK, _DPAD), jnp.float32),
            pltpu.SemaphoreType.DMA((2,)),
            pltpu.SemaphoreType.DMA((2,)),
        ],
        compiler_params=pltpu.CompilerParams(use_tc_tiling_on_sc=True),
    )
    def run(table_hbm, idx_hbm, out_hbm, idx_v, rows_v, gsem, ssem):
        wid = lax.axis_index("s") * _NC + lax.axis_index("c")
        base = wid * b_per_w
        pltpu.sync_copy(idx_hbm.at[wid], idx_v)

        def gather(buf, c):
            return pltpu.make_async_copy(
                table_hbm.at[idx_v.at[pl.ds(c * _CHUNK, _CHUNK)]],
                rows_v.at[buf], gsem.at[buf])

        def store(buf, c):
            return pltpu.make_async_copy(
                rows_v.at[buf], out_hbm.at[pl.ds(base + c * _CHUNK, _CHUNK)],
                ssem.at[buf])

        gather(0, 0).start()
        gather(1, 1).start()

        @pl.loop(0, n_pairs)
        def _pair(g):
            c0 = 2 * g
            c1 = c0 + 1
            last = n_chunks - 1
            c2 = jnp.minimum(c0 + 2, last)
            c3 = jnp.minimum(c0 + 3, last)
            gather(0, c0).wait()
            store(0, c0).start()
            gather(1, c1).wait()
            store(1, c1).start()
            store(0, c0).wait()
            gather(0, c2).start()
            store(1, c1).wait()
            gather(1, c3).start()

        gather(0, n_chunks - 1).wait()
        gather(1, n_chunks - 1).wait()

    return run(table, idx2d)


def kernel(idx, logits_table):
    nb, s = idx.shape
    v, d = logits_table.shape
    n = nb * s
    b_per_w = n // _NW
    row_pad = (b_per_w + _IPAD - 1) // _IPAD * _IPAD
    table = jnp.pad(logits_table, ((0, 0), (0, _DPAD - d)))
    flat = idx.reshape(_NW, b_per_w).astype(jnp.int32)
    idx2d = jnp.pad(flat, ((0, 0), (0, row_pad - b_per_w)))
    out = _gather_rows(table, idx2d, n)
    return out[:, :d].reshape(nb, s, d)


# speedup vs baseline: 2.0179x; 1.4624x over previous
import functools

import jax
import jax.numpy as jnp
from jax import lax
from jax.experimental import pallas as pl
from jax.experimental.pallas import tpu as pltpu
from jax.experimental.pallas import tpu_sc as plsc

_NC = 2
_NS = 16
_NW = _NC * _NS
_DPAD = 1024


@functools.partial(jax.jit, static_argnames=("nb", "s"))
def _gather_rows(table, idx3d, nb, s):
    b_per_w = nb // _NW
    assert b_per_w % 2 == 0
    n_pairs = b_per_w // 2
    mesh = plsc.VectorSubcoreMesh(
        core_axis_name="c", subcore_axis_name="s",
        num_cores=_NC, num_subcores=_NS)

    @functools.partial(
        pl.kernel,
        out_type=jax.ShapeDtypeStruct((nb, s, _DPAD), jnp.float32),
        mesh=mesh,
        scratch_types=[
            pltpu.VMEM((b_per_w, s), jnp.int32),
            pltpu.VMEM((2, s, _DPAD), jnp.float32),
            pltpu.SemaphoreType.DMA((2,)),
            pltpu.SemaphoreType.DMA((2,)),
        ],
        compiler_params=pltpu.CompilerParams(use_tc_tiling_on_sc=True),
    )
    def run(table_hbm, idx_hbm, out_hbm, idx_v, rows_v, gsem, ssem):
        wid = lax.axis_index("s") * _NC + lax.axis_index("c")
        base = wid * b_per_w
        pltpu.sync_copy(idx_hbm.at[wid], idx_v)

        def gather(buf, k):
            return pltpu.make_async_copy(
                table_hbm.at[idx_v.at[k]], rows_v.at[buf], gsem.at[buf])

        def store(buf, k):
            return pltpu.make_async_copy(
                rows_v.at[buf], out_hbm.at[base + k], ssem.at[buf])

        gather(0, 0).start()
        gather(1, 1).start()

        @pl.loop(0, n_pairs)
        def _pair(g):
            k0 = 2 * g
            k1 = k0 + 1
            last = b_per_w - 1
            k2 = jnp.minimum(k0 + 2, last)
            k3 = jnp.minimum(k0 + 3, last)
            gather(0, k0).wait()
            store(0, k0).start()
            gather(1, k1).wait()
            store(1, k1).start()
            store(0, k0).wait()
            gather(0, k2).start()
            store(1, k1).wait()
            gather(1, k3).start()

        gather(0, b_per_w - 1).wait()
        gather(1, b_per_w - 1).wait()

    return run(table, idx3d)


def kernel(idx, logits_table):
    nb, s = idx.shape
    v, d = logits_table.shape
    del v
    b_per_w = nb // _NW
    table = jnp.pad(logits_table, ((0, 0), (0, _DPAD - d)))
    idx3d = idx.reshape(_NW, b_per_w, s).astype(jnp.int32)
    out = _gather_rows(table, idx3d, nb, s)
    return out[:, :, :d]
